# Initial kernel scaffold; baseline (speedup 1.0000x reference)
import jax
import jax.numpy as jnp
from jax.experimental import pallas as pl


def kernel(logits, targets):
    raise NotImplementedError("write your pallas kernel here")


# speedup vs baseline: 21.6433x; 21.6433x over previous
import functools

import jax
import jax.numpy as jnp
from jax import lax
from jax.experimental import pallas as pl
from jax.experimental.pallas import tpu as pltpu
from jax.experimental.pallas import tpu_sc as plsc

M = 2048
N = 16 * 512 * 512
NC, NS, L = 2, 16, 16
NW = NC * NS
PER_W = N // NW
CHUNK = 4096
NCHUNK = PER_W // CHUNK
VPC = CHUNK // L

_mesh = functools.partial(
    plsc.VectorSubcoreMesh, core_axis_name="c", subcore_axis_name="s")


def _wid():
    return lax.axis_index("s") * NC + lax.axis_index("c")


def _all_sum(x):
    c = plsc.cumsum(x)
    r = lax.rev(plsc.cumsum(lax.rev(x, (0,))), (0,))
    return c + r - x


def _all_max(x):
    c = plsc.cummax(x)
    r = lax.rev(plsc.cummax(lax.rev(x, (0,))), (0,))
    return jnp.maximum(c, r)


@functools.partial(
    pl.kernel,
    out_type=jax.ShapeDtypeStruct((NW * L,), jnp.float32),
    mesh=_mesh(),
    compiler_params=pltpu.CompilerParams(needs_layout_passes=False),
    scratch_types=[pltpu.VMEM((CHUNK,), jnp.float32),
                   pltpu.VMEM((L,), jnp.float32)],
)
def _k1_maxabs(l_hbm, out_hbm, buf, stage):
    wid = _wid()
    base = wid * PER_W

    def chunk_body(i, mx):
        pltpu.sync_copy(l_hbm.at[pl.ds(base + i * CHUNK, CHUNK)], buf)

        def inner(j, mx):
            return jnp.maximum(mx, jnp.abs(buf[pl.ds(j * L, L)]))

        return lax.fori_loop(0, VPC, inner, mx)

    mx = lax.fori_loop(0, NCHUNK, chunk_body, jnp.zeros((L,), jnp.float32))
    stage[...] = mx
    pltpu.sync_copy(stage, out_hbm.at[pl.ds(wid * L, L)])


@functools.partial(
    pl.kernel,
    out_type=(jax.ShapeDtypeStruct((NW * M,), jnp.float32),
              jax.ShapeDtypeStruct((NW * M,), jnp.float32),
              jax.ShapeDtypeStruct((NW * M,), jnp.float32),
              jax.ShapeDtypeStruct((NW * L,), jnp.float32)),
    mesh=_mesh(),
    compiler_params=pltpu.CompilerParams(needs_layout_passes=False),
    scratch_types=[pltpu.VMEM((L * M,), jnp.int32),
                   pltpu.VMEM((L * M,), jnp.float32),
                   pltpu.VMEM((CHUNK,), jnp.float32),
                   pltpu.VMEM((CHUNK,), jnp.int32),
                   pltpu.VMEM((L * L * NC,), jnp.float32),
                   pltpu.VMEM((M,), jnp.float32),
                   pltpu.VMEM((M,), jnp.float32),
                   pltpu.VMEM((M,), jnp.float32),
                   pltpu.VMEM((L,), jnp.float32)],
)
def _k2_hist(l_hbm, t_hbm, maxv_hbm, cnt_hbm, pos_hbm, sume_hbm, psum_hbm,
             hist_pk, hist_se, lbuf, tbuf, mvbuf, cnt_row, pos_row, sume_row,
             psum_stage):
    wid = _wid()
    base = wid * PER_W

    pltpu.sync_copy(maxv_hbm, mvbuf)

    def mx_body(i, mx):
        return jnp.maximum(mx, mvbuf[pl.ds(i * L, L)])

    mxv = lax.fori_loop(0, NW * L // L, mx_body, jnp.zeros((L,), jnp.float32))
    scale = M / (1.0 + _all_max(mxv))

    zi = jnp.zeros((L,), jnp.int32)
    zf = jnp.zeros((L,), jnp.float32)

    def zero_body(k, _):
        hist_pk[pl.ds(k * L, L)] = zi
        hist_se[pl.ds(k * L, L)] = zf
        return 0

    lax.fori_loop(0, L * M // L, zero_body, 0)

    laneoff = lax.iota(jnp.int32, L) * M

    def chunk_body(i, pv):
        pltpu.sync_copy(l_hbm.at[pl.ds(base + i * CHUNK, CHUNK)], lbuf)
        pltpu.sync_copy(t_hbm.at[pl.ds(base + i * CHUNK, CHUNK)], tbuf)

        def inner(j, pv):
            lv = lbuf[pl.ds(j * L, L)]
            tv = tbuf[pl.ds(j * L, L)]
            tf = tv.astype(jnp.float32)
            e = 1.0 - lv * (2.0 * tf - 1.0)
            msk = e > 0.0
            b = jnp.minimum((e * scale).astype(jnp.int32), M - 1)
            addr = b + laneoff
            packed = (tv << 16) + 1
            plsc.addupdate_scatter(hist_pk, [addr], packed, mask=msk)
            plsc.addupdate_scatter(hist_se, [addr], e, mask=msk)
            return pv + tv

        return lax.fori_loop(0, VPC, inner, pv)

    pv = lax.fori_loop(0, NCHUNK, chunk_body, jnp.zeros((L,), jnp.int32))
    psum_stage[...] = pv.astype(jnp.float32)
    pltpu.sync_copy(psum_stage, psum_hbm.at[pl.ds(wid * L, L)])

    def red_body(v, _):
        accc = jnp.zeros((L,), jnp.int32)
        accp = jnp.zeros((L,), jnp.int32)
        accs = jnp.zeros((L,), jnp.float32)
        for lane in range(L):
            pk = hist_pk[pl.ds(lane * M + v * L, L)]
            accc = accc + (pk & 0xFFFF)
            accp = accp + (pk >> 16)
            accs = accs + hist_se[pl.ds(lane * M + v * L, L)]
        cnt_row[pl.ds(v * L, L)] = accc.astype(jnp.float32)
        pos_row[pl.ds(v * L, L)] = accp.astype(jnp.float32)
        sume_row[pl.ds(v * L, L)] = accs
        return 0

    lax.fori_loop(0, M // L, red_body, 0)
    pltpu.sync_copy(cnt_row, cnt_hbm.at[pl.ds(wid * M, M)])
    pltpu.sync_copy(pos_row, pos_hbm.at[pl.ds(wid * M, M)])
    pltpu.sync_copy(sume_row, sume_hbm.at[pl.ds(wid * M, M)])


@functools.partial(
    pl.kernel,
    out_type=jax.ShapeDtypeStruct((L,), jnp.float32),
    mesh=_mesh(),
    compiler_params=pltpu.CompilerParams(needs_layout_passes=False),
    scratch_types=[pltpu.VMEM((NW * M,), jnp.float32),
                   pltpu.VMEM((M,), jnp.float32),
                   pltpu.VMEM((M,), jnp.float32),
                   pltpu.VMEM((M,), jnp.float32),
                   pltpu.VMEM((NW * L,), jnp.float32),
                   pltpu.VMEM((L,), jnp.float32)],
)
def _k3_scan(cnt_hbm, pos_hbm, sume_hbm, psum_hbm, out_hbm,
             rows, gcnt, gpos, gse, pbuf, stage):
    wid = _wid()

    @pl.when(wid == 0)
    def _():
        pltpu.sync_copy(psum_hbm, pbuf)

        def p_body(i, acc):
            return acc + pbuf[pl.ds(i * L, L)]

        pv = lax.fori_loop(0, NW, p_body, jnp.zeros((L,), jnp.float32))
        P = _all_sum(pv)

        for src, dst in ((cnt_hbm, gcnt), (pos_hbm, gpos), (sume_hbm, gse)):
            pltpu.sync_copy(src, rows)

            def red_body(v, _, dst=dst):
                acc = jnp.zeros((L,), jnp.float32)
                for t in range(NW):
                    acc = acc + rows[pl.ds(t * M + v * L, L)]
                dst[pl.ds(v * L, L)] = acc
                return 0

            lax.fori_loop(0, M // L, red_body, 0)

        def jac(n, c):
            u = jnp.maximum(P + n - c, 1.0)
            j = 1.0 - (P - c) / u
            return jnp.where((n == 0.0) & (P == 0.0), 0.0, j)

        def scan_body(i, carry):
            lossv, nrun, crun = carry
            v = M // L - 1 - i
            dc = lax.rev(gcnt[pl.ds(v * L, L)], (0,))
            dp = lax.rev(gpos[pl.ds(v * L, L)], (0,))
            dsv = lax.rev(gse[pl.ds(v * L, L)], (0,))
            nincl = nrun + plsc.cumsum(dc)
            cincl = crun + plsc.cumsum(dp)
            nexcl = nincl - dc
            cexcl = cincl - dp
            dj = jac(nincl, cincl) - jac(nexcl, cexcl)
            lossv = lossv + (dsv / jnp.maximum(dc, 1.0)) * dj
            nrun = nrun + _all_sum(dc)
            crun = crun + _all_sum(dp)
            return lossv, nrun, crun

        lossv, _, _ = lax.fori_loop(
            0, M // L, scan_body,
            (jnp.zeros((L,), jnp.float32), jnp.zeros((L,), jnp.float32),
             jnp.zeros((L,), jnp.float32)))
        stage[...] = _all_sum(lossv)
        pltpu.sync_copy(stage, out_hbm)


def kernel(logits, targets):
    l = logits.reshape(N)
    t = targets.reshape(N)
    maxv = _k1_maxabs(l)
    cnt, pos, sume, psum = _k2_hist(l, t, maxv)
    out = _k3_scan(cnt, pos, sume, psum)
    return out[0]


# speedup vs baseline: 25.8556x; 1.1946x over previous
import functools

import jax
import jax.numpy as jnp
from jax import lax
from jax.experimental import pallas as pl
from jax.experimental.pallas import tpu as pltpu
from jax.experimental.pallas import tpu_sc as plsc

M = 2048
N = 16 * 512 * 512
NC, NS, L = 2, 16, 16
NW = NC * NS
PER_W = N // NW
CHUNK = 4096
NCHUNK = PER_W // CHUNK
VPC = CHUNK // L
U = 8

_mesh = functools.partial(
    plsc.VectorSubcoreMesh, core_axis_name="c", subcore_axis_name="s")


def _wid():
    return lax.axis_index("s") * NC + lax.axis_index("c")


def _all_sum(x):
    c = plsc.cumsum(x)
    r = lax.rev(plsc.cumsum(lax.rev(x, (0,))), (0,))
    return c + r - x


def _all_max(x):
    c = plsc.cummax(x)
    r = lax.rev(plsc.cummax(lax.rev(x, (0,))), (0,))
    return jnp.maximum(c, r)


@functools.partial(
    pl.kernel,
    out_type=jax.ShapeDtypeStruct((NW * L,), jnp.float32),
    mesh=_mesh(),
    compiler_params=pltpu.CompilerParams(needs_layout_passes=False),
    scratch_types=[pltpu.VMEM((CHUNK,), jnp.float32),
                   pltpu.VMEM((L,), jnp.float32)],
)
def _k1_maxabs(l_hbm, out_hbm, buf, stage):
    wid = _wid()
    base = wid * PER_W

    def chunk_body(i, mxs):
        pltpu.sync_copy(l_hbm.at[pl.ds(base + i * CHUNK, CHUNK)], buf)

        def inner(j, mxs):
            return tuple(
                jnp.maximum(mxs[u], jnp.abs(buf[pl.ds((j * U + u) * L, L)]))
                for u in range(U))

        return lax.fori_loop(0, VPC // U, inner, mxs)

    mxs = lax.fori_loop(0, NCHUNK, chunk_body,
                        tuple(jnp.zeros((L,), jnp.float32) for _ in range(U)))
    stage[...] = functools.reduce(jnp.maximum, mxs)
    pltpu.sync_copy(stage, out_hbm.at[pl.ds(wid * L, L)])


@functools.partial(
    pl.kernel,
    out_type=(jax.ShapeDtypeStruct((NW * M,), jnp.float32),
              jax.ShapeDtypeStruct((NW * M,), jnp.float32),
              jax.ShapeDtypeStruct((NW * M,), jnp.float32),
              jax.ShapeDtypeStruct((NW * L,), jnp.float32)),
    mesh=_mesh(),
    compiler_params=pltpu.CompilerParams(needs_layout_passes=False),
    scratch_types=[pltpu.VMEM((L * M,), jnp.int32),
                   pltpu.VMEM((L * M,), jnp.float32),
                   pltpu.VMEM((CHUNK,), jnp.float32),
                   pltpu.VMEM((CHUNK,), jnp.int32),
                   pltpu.VMEM((L * L * NC,), jnp.float32),
                   pltpu.VMEM((M,), jnp.float32),
                   pltpu.VMEM((M,), jnp.float32),
                   pltpu.VMEM((M,), jnp.float32),
                   pltpu.VMEM((L,), jnp.float32)],
)
def _k2_hist(l_hbm, t_hbm, maxv_hbm, cnt_hbm, pos_hbm, sume_hbm, psum_hbm,
             hist_pk, hist_se, lbuf, tbuf, mvbuf, cnt_row, pos_row, sume_row,
             psum_stage):
    wid = _wid()
    base = wid * PER_W

    pltpu.sync_copy(maxv_hbm, mvbuf)

    def mx_body(i, mx):
        return jnp.maximum(mx, mvbuf[pl.ds(i * L, L)])

    mxv = lax.fori_loop(0, NW * L // L, mx_body, jnp.zeros((L,), jnp.float32))
    scale = M / (1.0 + _all_max(mxv))

    zi = jnp.zeros((L,), jnp.int32)
    zf = jnp.zeros((L,), jnp.float32)

    def zero_body(k, _):
        hist_pk[pl.ds(k * L, L)] = zi
        hist_se[pl.ds(k * L, L)] = zf
        return 0

    lax.fori_loop(0, L * M // L, zero_body, 0)

    laneoff = lax.iota(jnp.int32, L) * M

    def chunk_body(i, pv):
        pltpu.sync_copy(l_hbm.at[pl.ds(base + i * CHUNK, CHUNK)], lbuf)
        pltpu.sync_copy(t_hbm.at[pl.ds(base + i * CHUNK, CHUNK)], tbuf)

        def inner(j, pv):
            for u in range(U):
                off = (j * U + u) * L
                lv = lbuf[pl.ds(off, L)]
                tv = tbuf[pl.ds(off, L)]
                mt = tv > 0
                e = jnp.where(mt, 1.0 - lv, 1.0 + lv)
                msk = e > 0.0
                bf = jnp.minimum(e * scale, float(M - 1))
                addr = bf.astype(jnp.int32) + laneoff
                val = jnp.where(mt, 0x10001, 1)
                plsc.addupdate_scatter(hist_pk, [addr], val, mask=msk)
                plsc.addupdate_scatter(hist_se, [addr], e, mask=msk)
                pv = pv + tv
            return pv

        return lax.fori_loop(0, VPC // U, inner, pv)

    pv = lax.fori_loop(0, NCHUNK, chunk_body, jnp.zeros((L,), jnp.int32))
    psum_stage[...] = pv.astype(jnp.float32)
    pltpu.sync_copy(psum_stage, psum_hbm.at[pl.ds(wid * L, L)])

    def red_body(v, _):
        accc = jnp.zeros((L,), jnp.int32)
        accp = jnp.zeros((L,), jnp.int32)
        accs = jnp.zeros((L,), jnp.float32)
        for lane in range(L):
            pk = hist_pk[pl.ds(lane * M + v * L, L)]
            accc = accc + (pk & 0xFFFF)
            accp = accp + (pk >> 16)
            accs = accs + hist_se[pl.ds(lane * M + v * L, L)]
        cnt_row[pl.ds(v * L, L)] = accc.astype(jnp.float32)
        pos_row[pl.ds(v * L, L)] = accp.astype(jnp.float32)
        sume_row[pl.ds(v * L, L)] = accs
        return 0

    lax.fori_loop(0, M // L, red_body, 0)
    pltpu.sync_copy(cnt_row, cnt_hbm.at[pl.ds(wid * M, M)])
    pltpu.sync_copy(pos_row, pos_hbm.at[pl.ds(wid * M, M)])
    pltpu.sync_copy(sume_row, sume_hbm.at[pl.ds(wid * M, M)])


@functools.partial(
    pl.kernel,
    out_type=jax.ShapeDtypeStruct((L,), jnp.float32),
    mesh=_mesh(),
    compiler_params=pltpu.CompilerParams(needs_layout_passes=False),
    scratch_types=[pltpu.VMEM((NW * M,), jnp.float32),
                   pltpu.VMEM((M,), jnp.float32),
                   pltpu.VMEM((M,), jnp.float32),
                   pltpu.VMEM((M,), jnp.float32),
                   pltpu.VMEM((NW * L,), jnp.float32),
                   pltpu.VMEM((L,), jnp.float32)],
)
def _k3_scan(cnt_hbm, pos_hbm, sume_hbm, psum_hbm, out_hbm,
             rows, gcnt, gpos, gse, pbuf, stage):
    wid = _wid()

    @pl.when(wid == 0)
    def _():
        pltpu.sync_copy(psum_hbm, pbuf)

        def p_body(i, acc):
            return acc + pbuf[pl.ds(i * L, L)]

        pv = lax.fori_loop(0, NW, p_body, jnp.zeros((L,), jnp.float32))
        P = _all_sum(pv)

        for src, dst in ((cnt_hbm, gcnt), (pos_hbm, gpos), (sume_hbm, gse)):
            pltpu.sync_copy(src, rows)

            def red_body(v, _, dst=dst):
                acc = jnp.zeros((L,), jnp.float32)
                for t in range(NW):
                    acc = acc + rows[pl.ds(t * M + v * L, L)]
                dst[pl.ds(v * L, L)] = acc
                return 0

            lax.fori_loop(0, M // L, red_body, 0)

        def jac(n, c):
            u = jnp.maximum(P + n - c, 1.0)
            j = 1.0 - (P - c) / u
            return jnp.where((n == 0.0) & (P == 0.0), 0.0, j)

        def scan_body(i, carry):
            lossv, nrun, crun = carry
            v = M // L - 1 - i
            dc = lax.rev(gcnt[pl.ds(v * L, L)], (0,))
            dp = lax.rev(gpos[pl.ds(v * L, L)], (0,))
            dsv = lax.rev(gse[pl.ds(v * L, L)], (0,))
            nincl = nrun + plsc.cumsum(dc)
            cincl = crun + plsc.cumsum(dp)
            nexcl = nincl - dc
            cexcl = cincl - dp
            dj = jac(nincl, cincl) - jac(nexcl, cexcl)
            lossv = lossv + (dsv / jnp.maximum(dc, 1.0)) * dj
            nrun = nrun + _all_sum(dc)
            crun = crun + _all_sum(dp)
            return lossv, nrun, crun

        lossv, _, _ = lax.fori_loop(
            0, M // L, scan_body,
            (jnp.zeros((L,), jnp.float32), jnp.zeros((L,), jnp.float32),
             jnp.zeros((L,), jnp.float32)))
        stage[...] = _all_sum(lossv)
        pltpu.sync_copy(stage, out_hbm)


def kernel(logits, targets):
    l = logits.reshape(N)
    t = targets.reshape(N)
    maxv = _k1_maxabs(l)
    cnt, pos, sume, psum = _k2_hist(l, t, maxv)
    out = _k3_scan(cnt, pos, sume, psum)
    return out[0]


# speedup vs baseline: 38.5623x; 1.4914x over previous
import functools

import jax
import jax.numpy as jnp
from jax import lax
from jax.experimental import pallas as pl
from jax.experimental.pallas import tpu as pltpu
from jax.experimental.pallas import tpu_sc as plsc

M = 2048
N = 16 * 512 * 512
NC, NS, L = 2, 16, 16
NW = NC * NS
PER_W = N // NW
CHUNK = 4096
NCHUNK = PER_W // CHUNK
VPC = CHUNK // L
U = 8

_mesh = functools.partial(
    plsc.VectorSubcoreMesh, core_axis_name="c", subcore_axis_name="s")


def _wid():
    return lax.axis_index("s") * NC + lax.axis_index("c")


def _all_sum(x):
    c = plsc.cumsum(x)
    r = lax.rev(plsc.cumsum(lax.rev(x, (0,))), (0,))
    return c + r - x


def _all_max(x):
    c = plsc.cummax(x)
    r = lax.rev(plsc.cummax(lax.rev(x, (0,))), (0,))
    return jnp.maximum(c, r)


@functools.partial(
    pl.kernel,
    out_type=jax.ShapeDtypeStruct((NW * L,), jnp.float32),
    mesh=_mesh(),
    compiler_params=pltpu.CompilerParams(needs_layout_passes=False),
    scratch_types=[pltpu.VMEM((CHUNK,), jnp.float32),
                   pltpu.VMEM((L,), jnp.float32)],
)
def _k1_maxabs(l_hbm, out_hbm, buf, stage):
    wid = _wid()
    base = wid * PER_W

    def chunk_body(i, mx):
        pltpu.sync_copy(l_hbm.at[pl.ds(base + i * CHUNK, CHUNK)], buf)

        @plsc.parallel_loop(0, VPC, carry=mx, unroll=U)
        def mx_out(j, mx):
            return jnp.maximum(mx, jnp.abs(buf[pl.ds(j * L, L)]))

        return mx_out

    stage[...] = lax.fori_loop(0, NCHUNK, chunk_body,
                               jnp.zeros((L,), jnp.float32))
    pltpu.sync_copy(stage, out_hbm.at[pl.ds(wid * L, L)])


@functools.partial(
    pl.kernel,
    out_type=(jax.ShapeDtypeStruct((NW * M,), jnp.float32),
              jax.ShapeDtypeStruct((NW * M,), jnp.float32),
              jax.ShapeDtypeStruct((NW * M,), jnp.float32),
              jax.ShapeDtypeStruct((NW * L,), jnp.float32)),
    mesh=_mesh(),
    compiler_params=pltpu.CompilerParams(needs_layout_passes=False),
    scratch_types=[pltpu.VMEM((L * M,), jnp.int32),
                   pltpu.VMEM((L * M,), jnp.float32),
                   pltpu.VMEM((CHUNK,), jnp.float32),
                   pltpu.VMEM((CHUNK,), jnp.int32),
                   pltpu.VMEM((L * L * NC,), jnp.float32),
                   pltpu.VMEM((M,), jnp.float32),
                   pltpu.VMEM((M,), jnp.float32),
                   pltpu.VMEM((M,), jnp.float32),
                   pltpu.VMEM((L,), jnp.float32)],
)
def _k2_hist(l_hbm, t_hbm, maxv_hbm, cnt_hbm, pos_hbm, sume_hbm, psum_hbm,
             hist_pk, hist_se, lbuf, tbuf, mvbuf, cnt_row, pos_row, sume_row,
             psum_stage):
    wid = _wid()
    base = wid * PER_W

    pltpu.sync_copy(maxv_hbm, mvbuf)

    def mx_body(i, mx):
        return jnp.maximum(mx, mvbuf[pl.ds(i * L, L)])

    mxv = lax.fori_loop(0, NW * L // L, mx_body, jnp.zeros((L,), jnp.float32))
    scale = M / (1.0 + _all_max(mxv))

    zi = jnp.zeros((L,), jnp.int32)
    zf = jnp.zeros((L,), jnp.float32)

    def zero_body(k, _):
        hist_pk[pl.ds(k * L, L)] = zi
        hist_se[pl.ds(k * L, L)] = zf
        return 0

    lax.fori_loop(0, L * M // L, zero_body, 0)

    laneoff = lax.iota(jnp.int32, L) * M

    def chunk_body(i, pv):
        pltpu.sync_copy(l_hbm.at[pl.ds(base + i * CHUNK, CHUNK)], lbuf)
        pltpu.sync_copy(t_hbm.at[pl.ds(base + i * CHUNK, CHUNK)], tbuf)

        @plsc.parallel_loop(0, VPC, carry=pv, unroll=U)
        def pv_out(j, pv):
            lv = lbuf[pl.ds(j * L, L)]
            tv = tbuf[pl.ds(j * L, L)]
            mt = tv > 0
            e = jnp.where(mt, 1.0 - lv, 1.0 + lv)
            msk = e > 0.0
            bf = jnp.minimum(e * scale, float(M - 1))
            addr = bf.astype(jnp.int32) + laneoff
            val = jnp.where(mt, 0x10001, 1)
            plsc.addupdate_scatter(hist_pk, [addr], val, mask=msk)
            plsc.addupdate_scatter(hist_se, [addr], e, mask=msk)
            return pv + tv

        return pv_out

    pv = lax.fori_loop(0, NCHUNK, chunk_body, jnp.zeros((L,), jnp.int32))
    psum_stage[...] = pv.astype(jnp.float32)
    pltpu.sync_copy(psum_stage, psum_hbm.at[pl.ds(wid * L, L)])

    def red_body(v, _):
        accc = jnp.zeros((L,), jnp.int32)
        accp = jnp.zeros((L,), jnp.int32)
        accs = jnp.zeros((L,), jnp.float32)
        for lane in range(L):
            pk = hist_pk[pl.ds(lane * M + v * L, L)]
            accc = accc + (pk & 0xFFFF)
            accp = accp + (pk >> 16)
            accs = accs + hist_se[pl.ds(lane * M + v * L, L)]
        cnt_row[pl.ds(v * L, L)] = accc.astype(jnp.float32)
        pos_row[pl.ds(v * L, L)] = accp.astype(jnp.float32)
        sume_row[pl.ds(v * L, L)] = accs
        return 0

    lax.fori_loop(0, M // L, red_body, 0)
    pltpu.sync_copy(cnt_row, cnt_hbm.at[pl.ds(wid * M, M)])
    pltpu.sync_copy(pos_row, pos_hbm.at[pl.ds(wid * M, M)])
    pltpu.sync_copy(sume_row, sume_hbm.at[pl.ds(wid * M, M)])


@functools.partial(
    pl.kernel,
    out_type=jax.ShapeDtypeStruct((L,), jnp.float32),
    mesh=_mesh(),
    compiler_params=pltpu.CompilerParams(needs_layout_passes=False),
    scratch_types=[pltpu.VMEM((NW * M,), jnp.float32),
                   pltpu.VMEM((M,), jnp.float32),
                   pltpu.VMEM((M,), jnp.float32),
                   pltpu.VMEM((M,), jnp.float32),
                   pltpu.VMEM((NW * L,), jnp.float32),
                   pltpu.VMEM((L,), jnp.float32)],
)
def _k3_scan(cnt_hbm, pos_hbm, sume_hbm, psum_hbm, out_hbm,
             rows, gcnt, gpos, gse, pbuf, stage):
    wid = _wid()

    @pl.when(wid == 0)
    def _():
        pltpu.sync_copy(psum_hbm, pbuf)

        def p_body(i, acc):
            return acc + pbuf[pl.ds(i * L, L)]

        pv = lax.fori_loop(0, NW, p_body, jnp.zeros((L,), jnp.float32))
        P = _all_sum(pv)

        for src, dst in ((cnt_hbm, gcnt), (pos_hbm, gpos), (sume_hbm, gse)):
            pltpu.sync_copy(src, rows)

            def red_body(v, _, dst=dst):
                acc = jnp.zeros((L,), jnp.float32)
                for t in range(NW):
                    acc = acc + rows[pl.ds(t * M + v * L, L)]
                dst[pl.ds(v * L, L)] = acc
                return 0

            lax.fori_loop(0, M // L, red_body, 0)

        def jac(n, c):
            u = jnp.maximum(P + n - c, 1.0)
            j = 1.0 - (P - c) / u
            return jnp.where((n == 0.0) & (P == 0.0), 0.0, j)

        def scan_body(i, carry):
            lossv, nrun, crun = carry
            v = M // L - 1 - i
            dc = lax.rev(gcnt[pl.ds(v * L, L)], (0,))
            dp = lax.rev(gpos[pl.ds(v * L, L)], (0,))
            dsv = lax.rev(gse[pl.ds(v * L, L)], (0,))
            nincl = nrun + plsc.cumsum(dc)
            cincl = crun + plsc.cumsum(dp)
            nexcl = nincl - dc
            cexcl = cincl - dp
            dj = jac(nincl, cincl) - jac(nexcl, cexcl)
            lossv = lossv + (dsv / jnp.maximum(dc, 1.0)) * dj
            nrun = nrun + _all_sum(dc)
            crun = crun + _all_sum(dp)
            return lossv, nrun, crun

        lossv, _, _ = lax.fori_loop(
            0, M // L, scan_body,
            (jnp.zeros((L,), jnp.float32), jnp.zeros((L,), jnp.float32),
             jnp.zeros((L,), jnp.float32)))
        stage[...] = _all_sum(lossv)
        pltpu.sync_copy(stage, out_hbm)


def kernel(logits, targets):
    l = logits.reshape(N)
    t = targets.reshape(N)
    maxv = _k1_maxabs(l)
    cnt, pos, sume, psum = _k2_hist(l, t, maxv)
    out = _k3_scan(cnt, pos, sume, psum)
    return out[0]
